# Initial kernel scaffold; baseline (speedup 1.0000x reference)
import jax
import jax.numpy as jnp
from jax.experimental import pallas as pl


def kernel(x, edge_index, batch, Wl1, Wr1, b1, g1, be1, Wl2, Wr2, b2, gm1, bm1, Wlin1, blin1, gm2, bm2, Wlin2, blin2, gm3, bm3, Wlin3, blin3, Wout, bout):
    raise NotImplementedError("write your pallas kernel here")


# speedup vs baseline: 13.0210x; 13.0210x over previous
import functools

import jax
import jax.numpy as jnp
from jax import lax
from jax.experimental import pallas as pl
from jax.experimental.pallas import tpu as pltpu
from jax.experimental.pallas import tpu_sc as plsc

_N = 10000
_E = 640000
_D = 128
_H = 16
_G = 64
_EB = 128
_NB = _E // _EB
_NC = 2
_NS = 16
_NW = _NC * _NS
_RPT = _N // _NS


def _proj_body(x_ref, w_ref, y_ref, r_ref):
    res = jnp.dot(x_ref[...], w_ref[...], preferred_element_type=jnp.float32)
    y_ref[...] = res[:, :_H]
    r_ref[...] = res[:, _H:]


def _project(x, w):
    n, k = x.shape
    bn = 1000
    return pl.pallas_call(
        _proj_body,
        grid=(n // bn,),
        in_specs=[
            pl.BlockSpec((bn, k), lambda i: (i, 0)),
            pl.BlockSpec((k, 2 * _H), lambda i: (0, 0)),
        ],
        out_specs=[
            pl.BlockSpec((bn, _H), lambda i: (i, 0)),
            pl.BlockSpec((bn, _H), lambda i: (i, 0)),
        ],
        out_shape=[
            jax.ShapeDtypeStruct((n, _H), jnp.float32),
            jax.ShapeDtypeStruct((n, _H), jnp.float32),
        ],
    )(x, w)


@functools.lru_cache(maxsize=None)
def _make_edge_pass(with_cnt: bool):
    mesh = plsc.VectorSubcoreMesh(core_axis_name="c", subcore_axis_name="s",
                                  num_cores=_NC, num_subcores=_NS)

    out_type = [jax.ShapeDtypeStruct((_NC, _N, _H), jnp.float32)]
    scratch = [
        pltpu.VMEM_SHARED((_N, _H), jnp.float32),
        pltpu.VMEM((_EB,), jnp.int32),
        pltpu.VMEM((_EB,), jnp.int32),
        pltpu.VMEM((_EB, _H), jnp.float32),
        pltpu.SemaphoreType.DMA,
    ]
    if with_cnt:
        out_type.append(jax.ShapeDtypeStruct((_NC, _N), jnp.float32))
        scratch += [
            pltpu.VMEM_SHARED((_N,), jnp.float32),
            pltpu.VMEM((_EB,), jnp.float32),
        ]

    def body(y_hbm, src_hbm, dst_hbm, z2_hbm, z1_hbm, *rest):
        if with_cnt:
            acc_out, cnt_out, acc_sh, sbuf, dbuf, rows, sem, cnt_sh, ones_v = rest
        else:
            acc_out, acc_sh, sbuf, dbuf, rows, sem = rest
        cid = lax.axis_index("c")
        sid = lax.axis_index("s")
        wid = sid * _NC + cid

        pltpu.sync_copy(z2_hbm.at[pl.ds(sid * _RPT, _RPT)],
                        acc_sh.at[pl.ds(sid * _RPT, _RPT)])
        if with_cnt:
            @pl.when(sid < 10)
            def _():
                pltpu.sync_copy(z1_hbm.at[pl.ds(sid * 1000, 1000)],
                                cnt_sh.at[pl.ds(sid * 1000, 1000)])
            for i in range(_EB // 16):
                ones_v[pl.ds(i * 16, 16)] = jnp.ones((16,), jnp.float32)
        plsc.subcore_barrier()

        nloops = (_NB + _NW - 1) // _NW

        def step(j, carry):
            r = j * _NW + wid

            @pl.when(r < _NB)
            def _():
                pltpu.sync_copy(src_hbm.at[r], sbuf)
                pltpu.sync_copy(dst_hbm.at[r], dbuf)
                pltpu.async_copy(y_hbm.at[sbuf], rows, sem).wait()
                pltpu.sync_copy(rows, acc_sh.at[dbuf], add=True)
                if with_cnt:
                    pltpu.sync_copy(ones_v, cnt_sh.at[dbuf], add=True)
            return carry

        lax.fori_loop(0, nloops, step, 0)
        plsc.subcore_barrier()

        pltpu.sync_copy(acc_sh.at[pl.ds(sid * _RPT, _RPT)],
                        acc_out.at[cid, pl.ds(sid * _RPT, _RPT)])
        if with_cnt:
            @pl.when(sid < 10)
            def _():
                pltpu.sync_copy(cnt_sh.at[pl.ds(sid * 1000, 1000)],
                                cnt_out.at[cid, pl.ds(sid * 1000, 1000)])

    return pl.kernel(
        body, out_type=out_type, mesh=mesh, scratch_types=scratch,
        compiler_params=pltpu.CompilerParams(use_tc_tiling_on_sc=False))


def _edge_pass_cnt(*args):
    return _make_edge_pass(True)(*args)


def _edge_pass(*args):
    return _make_edge_pass(False)(*args)


def _mid_body(a0, a1, c0, c1, r1, b1r, g1r, be1r, w2, y2_o, r2_o, cnt_o):
    c = c0[...] + c1[...]
    s = a0[...] + a1[...]
    mean = s / jnp.maximum(c, 1.0)
    h = jnp.maximum(mean + b1r[...] + r1[...], 0.0)
    m = jnp.mean(h, axis=0, keepdims=True)
    v = jnp.mean((h - m) ** 2, axis=0, keepdims=True)
    hn = (h - m) / jnp.sqrt(v + 1e-5) * g1r[...] + be1r[...]
    res = jnp.dot(hn, w2[...], preferred_element_type=jnp.float32)
    y2_o[...] = res[:, :_H]
    r2_o[...] = res[:, _H:]
    cnt_o[...] = c


def _mid(a0, a1, c0, c1, r1, b1r, g1r, be1r, w2):
    return pl.pallas_call(
        _mid_body,
        out_shape=[
            jax.ShapeDtypeStruct((_N, _H), jnp.float32),
            jax.ShapeDtypeStruct((_N, _H), jnp.float32),
            jax.ShapeDtypeStruct((_N, 1), jnp.float32),
        ],
    )(a0, a1, c0, c1, r1, b1r, g1r, be1r, w2)


def _bn64(x, g, b):
    m = jnp.mean(x, axis=0, keepdims=True)
    v = jnp.mean((x - m) ** 2, axis=0, keepdims=True)
    return (x - m) / jnp.sqrt(v + 1e-5) * g + b


def _fin_body(a0, a1, cnt, r2, b2r, batch_r, gm1r, bm1r, wl1, bl1, gm2r, bm2r,
              wl2, bl2, gm3r, bm3r, wl3, bl3, wo, bo, out_o):
    c = cnt[...]
    mean = (a0[...] + a1[...]) / jnp.maximum(c, 1.0)
    h2 = jnp.maximum(mean + b2r[...] + r2[...], 0.0)
    gids = lax.broadcasted_iota(jnp.int32, (_G, _N), 0)
    oh = (gids == batch_r[...]).astype(jnp.float32)
    s = jnp.dot(oh, h2, preferred_element_type=jnp.float32)
    cg = jnp.sum(oh, axis=1, keepdims=True)
    xe = s / jnp.maximum(cg, 1.0)

    h = _bn64(xe, gm1r[...], bm1r[...])
    h = jnp.maximum(jnp.dot(h, wl1[...], preferred_element_type=jnp.float32)
                    + bl1[...], 0.0)
    h = _bn64(jnp.concatenate([h, xe], axis=1), gm2r[...], bm2r[...])
    h = jnp.maximum(jnp.dot(h, wl2[...], preferred_element_type=jnp.float32)
                    + bl2[...], 0.0)
    h = _bn64(jnp.concatenate([h, xe], axis=1), gm3r[...], bm3r[...])
    h = jnp.maximum(jnp.dot(h, wl3[...], preferred_element_type=jnp.float32)
                    + bl3[...], 0.0)
    out_o[...] = jnp.dot(h, wo[...], preferred_element_type=jnp.float32) + bo[...]


def _fin(*args):
    return pl.pallas_call(
        _fin_body,
        out_shape=jax.ShapeDtypeStruct((_G, 1), jnp.float32),
    )(*args)


def kernel(x, edge_index, batch, Wl1, Wr1, b1, g1, be1, Wl2, Wr2, b2, gm1, bm1,
           Wlin1, blin1, gm2, bm2, Wlin2, blin2, gm3, bm3, Wlin3, blin3, Wout,
           bout):
    f32 = jnp.float32
    src = edge_index[0].reshape(_NB, _EB)
    dst = edge_index[1].reshape(_NB, _EB)
    z2 = jnp.zeros((_N, _H), f32)
    z1 = jnp.zeros((_N,), f32)

    w1 = jnp.concatenate([Wl1, Wr1], axis=1)
    y1, r1 = _project(x, w1)

    acc1, cnt = _edge_pass_cnt(y1, src, dst, z2, z1)

    w2 = jnp.concatenate([Wl2, Wr2], axis=1)
    y2, r2, cntc = _mid(acc1[0], acc1[1],
                        cnt[0].reshape(_N, 1), cnt[1].reshape(_N, 1),
                        r1, b1.reshape(1, _H), g1.reshape(1, _H),
                        be1.reshape(1, _H), w2)

    acc2 = (_edge_pass(y2, src, dst, z2, z1))[0]

    out = _fin(acc2[0], acc2[1], cntc, r2, b2.reshape(1, _H),
               batch.reshape(1, _N).astype(jnp.int32),
               gm1.reshape(1, _H), bm1.reshape(1, _H), Wlin1,
               blin1.reshape(1, _H), gm2.reshape(1, 2 * _H),
               bm2.reshape(1, 2 * _H), Wlin2, blin2.reshape(1, _H),
               gm3.reshape(1, 2 * _H), bm3.reshape(1, 2 * _H), Wlin3,
               blin3.reshape(1, _H), Wout, bout.reshape(1, 1))
    return out


# speedup vs baseline: 19.9867x; 1.5350x over previous
import functools

import jax
import jax.numpy as jnp
from jax import lax
from jax.experimental import pallas as pl
from jax.experimental.pallas import tpu as pltpu
from jax.experimental.pallas import tpu_sc as plsc

_N = 10000
_E = 640000
_D = 128
_H = 16
_G = 64
_EB = 128
_NC = 2
_NS = 16
_NW = _NC * _NS
_BPW = 160
_EPAD = _NW * _BPW * _EB
_NPAD = 10240
_GB = 8
_NGRP = _BPW // _GB
_NPAIR = _NGRP // 2


def _proj_body(x_ref, w_ref, y_ref, r_ref):
    res = jnp.dot(x_ref[...], w_ref[...], preferred_element_type=jnp.float32)
    y_ref[...] = res[:, :_H]
    r_ref[...] = res[:, _H:]


def _project(x, w):
    n, k = x.shape
    bn = 1000
    return pl.pallas_call(
        _proj_body,
        grid=(n // bn,),
        in_specs=[
            pl.BlockSpec((bn, k), lambda i: (i, 0)),
            pl.BlockSpec((k, 2 * _H), lambda i: (0, 0)),
        ],
        out_specs=[
            pl.BlockSpec((bn, _H), lambda i: (i, 0)),
            pl.BlockSpec((bn, _H), lambda i: (i, 0)),
        ],
        out_shape=[
            jax.ShapeDtypeStruct((n, _H), jnp.float32),
            jax.ShapeDtypeStruct((n, _H), jnp.float32),
        ],
    )(x, w)


@functools.lru_cache(maxsize=None)
def _make_edge_pass(with_cnt: bool):
    mesh = plsc.VectorSubcoreMesh(core_axis_name="c", subcore_axis_name="s",
                                  num_cores=_NC, num_subcores=_NS)
    out_type = [jax.ShapeDtypeStruct((_NC, _NPAD, _H), jnp.float32)]
    scratch = [
        pltpu.VMEM_SHARED((_NPAD, _H), jnp.float32),
        pltpu.VMEM((_GB, _EB), jnp.int32),
        pltpu.VMEM((_GB, _EB), jnp.int32),
        pltpu.VMEM((_GB, _EB), jnp.int32),
        pltpu.VMEM((_GB, _EB), jnp.int32),
        pltpu.VMEM((_GB, _EB, _H), jnp.float32),
        pltpu.VMEM((_GB, _EB, _H), jnp.float32),
        pltpu.SemaphoreType.DMA,
        pltpu.SemaphoreType.DMA,
        pltpu.SemaphoreType.DMA,
        pltpu.SemaphoreType.DMA,
    ]
    if with_cnt:
        out_type.append(jax.ShapeDtypeStruct((_NC, _NPAD), jnp.float32))
        scratch += [
            pltpu.VMEM_SHARED((_NPAD,), jnp.float32),
            pltpu.VMEM((_EB,), jnp.float32),
        ]
    st = _NPAD // _NS

    def body(y_hbm, src_hbm, dst_hbm, z_hbm, z1_hbm, *rest):
        if with_cnt:
            (acc_out, cnt_out, acc_sh, sbufA, sbufB, dbufA, dbufB, rowsA,
             rowsB, semGA, semGB, semS, semSL, cnt_sh, ones_v) = rest
        else:
            (acc_out, acc_sh, sbufA, sbufB, dbufA, dbufB, rowsA, rowsB,
             semGA, semGB, semS, semSL) = rest
        cid = lax.axis_index("c")
        sid = lax.axis_index("s")
        wid = sid * _NC + cid

        pltpu.sync_copy(z_hbm.at[pl.ds(sid * st, st)],
                        acc_sh.at[pl.ds(sid * st, st)])
        if with_cnt:
            pltpu.sync_copy(z1_hbm.at[pl.ds(sid * st, st)],
                            cnt_sh.at[pl.ds(sid * st, st)])
            for i in range(_EB // 16):
                ones_v[pl.ds(i * 16, 16)] = jnp.ones((16,), jnp.float32)

        def load_slabs_sync(g, sb, db):
            pltpu.sync_copy(src_hbm.at[wid, pl.ds(g * _GB, _GB)], sb)
            pltpu.sync_copy(dst_hbm.at[wid, pl.ds(g * _GB, _GB)], db)

        def fire_gathers(sb, rows, sem):
            for b in range(_GB):
                pltpu.async_copy(y_hbm.at[sb.at[b]], rows.at[b], sem)

        def drain_gathers(sb, rows, sem):
            for b in range(_GB):
                pltpu.make_async_copy(y_hbm.at[sb.at[b]], rows.at[b],
                                      sem).wait()

        def fire_scatters(db, rows):
            for b in range(_GB):
                pltpu.async_copy(rows.at[b], acc_sh.at[db.at[b]], semS,
                                 add=True)

        def drain_scatters(db, rows):
            for b in range(_GB):
                pltpu.make_async_copy(rows.at[b], acc_sh.at[db.at[b]],
                                      semS).wait()

        plsc.subcore_barrier()

        def group(t, carry):
            load_slabs_sync(t, sbufA, dbufA)
            fire_gathers(sbufA, rowsA, semGA)
            drain_gathers(sbufA, rowsA, semGA)
            for b in range(_GB):
                pltpu.sync_copy(rowsA.at[b], acc_sh.at[dbufA.at[b]], add=True)
                if with_cnt:
                    pltpu.sync_copy(ones_v, cnt_sh.at[dbufA.at[b]], add=True)
            return carry

        lax.fori_loop(0, _NGRP, group, 0)
        plsc.subcore_barrier()

        pltpu.sync_copy(acc_sh.at[pl.ds(sid * st, st)],
                        acc_out.at[cid, pl.ds(sid * st, st)])
        if with_cnt:
            pltpu.sync_copy(cnt_sh.at[pl.ds(sid * st, st)],
                            cnt_out.at[cid, pl.ds(sid * st, st)])

    return pl.kernel(
        body, out_type=out_type, mesh=mesh, scratch_types=scratch,
        compiler_params=pltpu.CompilerParams(use_tc_tiling_on_sc=False))


def _edge_pass(with_cnt, *args):
    return _make_edge_pass(with_cnt)(*args)


def _mid_body(acc, cnt, r1, b1r, g1r, be1r, w2, y2_o, r2_o, cnt_o):
    s = acc[0, :_N, :] + acc[1, :_N, :]
    c = cnt[0, :_N, :] + cnt[1, :_N, :]
    mean = s / jnp.maximum(c, 1.0)
    h = jnp.maximum(mean + b1r[...] + r1[...], 0.0)
    m = jnp.mean(h, axis=0, keepdims=True)
    v = jnp.mean((h - m) ** 2, axis=0, keepdims=True)
    hn = (h - m) / jnp.sqrt(v + 1e-5) * g1r[...] + be1r[...]
    res = jnp.dot(hn, w2[...], preferred_element_type=jnp.float32)
    y2_o[...] = res[:, :_H]
    r2_o[...] = res[:, _H:]
    cnt_o[...] = c


def _mid(acc, cnt, r1, b1r, g1r, be1r, w2):
    return pl.pallas_call(
        _mid_body,
        out_shape=[
            jax.ShapeDtypeStruct((_N, _H), jnp.float32),
            jax.ShapeDtypeStruct((_N, _H), jnp.float32),
            jax.ShapeDtypeStruct((_N, 1), jnp.float32),
        ],
    )(acc, cnt, r1, b1r, g1r, be1r, w2)


def _bn64(x, g, b):
    m = jnp.mean(x, axis=0, keepdims=True)
    v = jnp.mean((x - m) ** 2, axis=0, keepdims=True)
    return (x - m) / jnp.sqrt(v + 1e-5) * g + b


def _fin_body(acc, cnt, r2, b2r, batch_r, gm1r, bm1r, wl1, bl1, gm2r, bm2r,
              wl2, bl2, gm3r, bm3r, wl3, bl3, wo, bo, out_o):
    c = cnt[...]
    mean = (acc[0, :_N, :] + acc[1, :_N, :]) / jnp.maximum(c, 1.0)
    h2 = jnp.maximum(mean + b2r[...] + r2[...], 0.0)
    gids = lax.broadcasted_iota(jnp.int32, (_G, _N), 0)
    oh = (gids == batch_r[...]).astype(jnp.float32)
    s = jnp.dot(oh, h2, preferred_element_type=jnp.float32)
    cg = jnp.sum(oh, axis=1, keepdims=True)
    xe = s / jnp.maximum(cg, 1.0)

    h = _bn64(xe, gm1r[...], bm1r[...])
    h = jnp.maximum(jnp.dot(h, wl1[...], preferred_element_type=jnp.float32)
                    + bl1[...], 0.0)
    h = _bn64(jnp.concatenate([h, xe], axis=1), gm2r[...], bm2r[...])
    h = jnp.maximum(jnp.dot(h, wl2[...], preferred_element_type=jnp.float32)
                    + bl2[...], 0.0)
    h = _bn64(jnp.concatenate([h, xe], axis=1), gm3r[...], bm3r[...])
    h = jnp.maximum(jnp.dot(h, wl3[...], preferred_element_type=jnp.float32)
                    + bl3[...], 0.0)
    out_o[...] = jnp.dot(h, wo[...], preferred_element_type=jnp.float32) + bo[...]


def _fin(*args):
    return pl.pallas_call(
        _fin_body,
        out_shape=jax.ShapeDtypeStruct((_G, 1), jnp.float32),
    )(*args)


def kernel(x, edge_index, batch, Wl1, Wr1, b1, g1, be1, Wl2, Wr2, b2, gm1, bm1,
           Wlin1, blin1, gm2, bm2, Wlin2, blin2, gm3, bm3, Wlin3, blin3, Wout,
           bout):
    f32 = jnp.float32
    pad = _EPAD - _E
    srcp = jnp.concatenate(
        [edge_index[0], jnp.zeros((pad,), jnp.int32)]).reshape(_NW, _BPW, _EB)
    dstp = jnp.concatenate(
        [edge_index[1], jnp.full((pad,), _N, jnp.int32)]).reshape(_NW, _BPW, _EB)
    z16 = jnp.zeros((_NPAD, _H), f32)
    z1 = jnp.zeros((_NPAD,), f32)

    w1 = jnp.concatenate([Wl1, Wr1], axis=1)
    y1, r1 = _project(x, w1)

    acc1, cnt = _edge_pass(True, y1, srcp, dstp, z16, z1)

    w2 = jnp.concatenate([Wl2, Wr2], axis=1)
    y2, r2, cntc = _mid(acc1, cnt.reshape(_NC, _NPAD, 1), r1,
                        b1.reshape(1, _H), g1.reshape(1, _H),
                        be1.reshape(1, _H), w2)

    acc2 = _edge_pass(False, y2, srcp, dstp, z16, z1)[0]

    out = _fin(acc2, cntc, r2, b2.reshape(1, _H),
               batch.reshape(1, _N).astype(jnp.int32),
               gm1.reshape(1, _H), bm1.reshape(1, _H), Wlin1,
               blin1.reshape(1, _H), gm2.reshape(1, 2 * _H),
               bm2.reshape(1, 2 * _H), Wlin2, blin2.reshape(1, _H),
               gm3.reshape(1, 2 * _H), bm3.reshape(1, 2 * _H), Wlin3,
               blin3.reshape(1, _H), Wout, bout.reshape(1, 1))
    return out


# speedup vs baseline: 23.7463x; 1.1881x over previous
import functools

import jax
import jax.numpy as jnp
from jax import lax
from jax.experimental import pallas as pl
from jax.experimental.pallas import tpu as pltpu
from jax.experimental.pallas import tpu_sc as plsc

_N = 10000
_E = 640000
_D = 128
_H = 16
_G = 64
_EB = 128
_NC = 2
_NS = 16
_NW = _NC * _NS
_BPW = 160
_EPAD = _NW * _BPW * _EB
_NPAD = 10240
_GB = 8
_NGRP = _BPW // _GB
_NPAIR = _NGRP // 2


def _proj_body(x_ref, w_ref, y_ref, r_ref):
    res = jnp.dot(x_ref[...], w_ref[...], preferred_element_type=jnp.float32)
    y_ref[...] = res[:, :_H]
    r_ref[...] = res[:, _H:]


def _project(x, w):
    n, k = x.shape
    bn = 1000
    return pl.pallas_call(
        _proj_body,
        grid=(n // bn,),
        in_specs=[
            pl.BlockSpec((bn, k), lambda i: (i, 0)),
            pl.BlockSpec((k, 2 * _H), lambda i: (0, 0)),
        ],
        out_specs=[
            pl.BlockSpec((bn, _H), lambda i: (i, 0)),
            pl.BlockSpec((bn, _H), lambda i: (i, 0)),
        ],
        out_shape=[
            jax.ShapeDtypeStruct((n, _H), jnp.float32),
            jax.ShapeDtypeStruct((n, _H), jnp.float32),
        ],
    )(x, w)


@functools.lru_cache(maxsize=None)
def _make_edge_pass(with_cnt: bool):
    mesh = plsc.VectorSubcoreMesh(core_axis_name="c", subcore_axis_name="s",
                                  num_cores=_NC, num_subcores=_NS)
    out_type = [jax.ShapeDtypeStruct((_NC, _NPAD, _H), jnp.float32)]
    scratch = [
        pltpu.VMEM_SHARED((_NPAD, _H), jnp.float32),
        pltpu.VMEM((_GB, _EB), jnp.int32),
        pltpu.VMEM((_GB, _EB), jnp.int32),
        pltpu.VMEM((_GB, _EB), jnp.int32),
        pltpu.VMEM((_GB, _EB), jnp.int32),
        pltpu.VMEM((_GB, _EB, _H), jnp.float32),
        pltpu.VMEM((_GB, _EB, _H), jnp.float32),
        pltpu.SemaphoreType.DMA,
        pltpu.SemaphoreType.DMA,
        pltpu.SemaphoreType.DMA,
        pltpu.SemaphoreType.DMA,
    ]
    if with_cnt:
        out_type.append(jax.ShapeDtypeStruct((_NC, _NPAD), jnp.float32))
        scratch += [
            pltpu.VMEM_SHARED((_NPAD,), jnp.float32),
            pltpu.VMEM((_EB,), jnp.float32),
        ]
    st = _NPAD // _NS

    def body(y_hbm, src_hbm, dst_hbm, z_hbm, z1_hbm, *rest):
        if with_cnt:
            (acc_out, cnt_out, acc_sh, sbufA, sbufB, dbufA, dbufB, rowsA,
             rowsB, semGA, semGB, semS, semSL, cnt_sh, ones_v) = rest
        else:
            (acc_out, acc_sh, sbufA, sbufB, dbufA, dbufB, rowsA, rowsB,
             semGA, semGB, semS, semSL) = rest
        cid = lax.axis_index("c")
        sid = lax.axis_index("s")
        wid = sid * _NC + cid

        pltpu.sync_copy(z_hbm.at[pl.ds(sid * st, st)],
                        acc_sh.at[pl.ds(sid * st, st)])
        if with_cnt:
            pltpu.sync_copy(z1_hbm.at[pl.ds(sid * st, st)],
                            cnt_sh.at[pl.ds(sid * st, st)])
            for i in range(_EB // 16):
                ones_v[pl.ds(i * 16, 16)] = jnp.ones((16,), jnp.float32)

        def load_slabs_sync(g, sb, db):
            pltpu.sync_copy(src_hbm.at[wid, pl.ds(g * _GB, _GB)], sb)
            pltpu.sync_copy(dst_hbm.at[wid, pl.ds(g * _GB, _GB)], db)

        def fire_gathers(sb, rows, sem):
            for b in range(_GB):
                pltpu.async_copy(y_hbm.at[sb.at[b]], rows.at[b], sem)

        def drain_gathers(sb, rows, sem):
            for b in range(_GB):
                pltpu.make_async_copy(y_hbm.at[sb.at[b]], rows.at[b],
                                      sem).wait()

        def fire_scatters(db, rows):
            for b in range(_GB):
                pltpu.async_copy(rows.at[b], acc_sh.at[db.at[b]], semS,
                                 add=True)

        def drain_scatters(db, rows):
            for b in range(_GB):
                pltpu.make_async_copy(rows.at[b], acc_sh.at[db.at[b]],
                                      semS).wait()

        plsc.subcore_barrier()

        def fire_scatters(db, rows):
            for b in range(_GB):
                pltpu.async_copy(rows.at[b], acc_sh.at[db.at[b]], semS,
                                 add=True)
                if with_cnt:
                    pltpu.async_copy(ones_v, cnt_sh.at[db.at[b]], semS,
                                     add=True)

        def drain_scatters(db, rows):
            for b in range(_GB):
                pltpu.make_async_copy(rows.at[b], acc_sh.at[db.at[b]],
                                      semS).wait()
                if with_cnt:
                    pltpu.make_async_copy(ones_v, cnt_sh.at[db.at[b]],
                                          semS).wait()

        load_slabs_sync(0, sbufA, dbufA)
        fire_gathers(sbufA, rowsA, semGA)
        load_slabs_sync(1, sbufB, dbufB)

        def pair(t, carry):
            drain_gathers(sbufA, rowsA, semGA)
            fire_gathers(sbufB, rowsB, semGB)
            fire_scatters(dbufA, rowsA)
            drain_scatters(dbufA, rowsA)

            @pl.when(t < _NPAIR - 1)
            def _():
                pltpu.async_copy(src_hbm.at[wid, pl.ds((2 * t + 2) * _GB, _GB)],
                                 sbufA, semSL)
                pltpu.async_copy(dst_hbm.at[wid, pl.ds((2 * t + 2) * _GB, _GB)],
                                 dbufA, semSL)

            drain_gathers(sbufB, rowsB, semGB)
            fire_scatters(dbufB, rowsB)
            drain_scatters(dbufB, rowsB)

            @pl.when(t < _NPAIR - 1)
            def _():
                pltpu.make_async_copy(src_hbm.at[wid, pl.ds(0, _GB)], sbufA,
                                      semSL).wait()
                pltpu.make_async_copy(dst_hbm.at[wid, pl.ds(0, _GB)], dbufA,
                                      semSL).wait()
                fire_gathers(sbufA, rowsA, semGA)
                load_slabs_sync(2 * t + 3, sbufB, dbufB)
            return carry

        lax.fori_loop(0, _NPAIR, pair, 0)
        plsc.subcore_barrier()

        pltpu.sync_copy(acc_sh.at[pl.ds(sid * st, st)],
                        acc_out.at[cid, pl.ds(sid * st, st)])
        if with_cnt:
            pltpu.sync_copy(cnt_sh.at[pl.ds(sid * st, st)],
                            cnt_out.at[cid, pl.ds(sid * st, st)])

    return pl.kernel(
        body, out_type=out_type, mesh=mesh, scratch_types=scratch,
        compiler_params=pltpu.CompilerParams(use_tc_tiling_on_sc=False))


def _edge_pass(with_cnt, *args):
    return _make_edge_pass(with_cnt)(*args)


def _mid_body(acc, cnt, r1, b1r, g1r, be1r, w2, y2_o, r2_o, cnt_o):
    s = acc[0, :_N, :] + acc[1, :_N, :]
    c = cnt[0, :_N, :] + cnt[1, :_N, :]
    mean = s / jnp.maximum(c, 1.0)
    h = jnp.maximum(mean + b1r[...] + r1[...], 0.0)
    m = jnp.mean(h, axis=0, keepdims=True)
    v = jnp.mean((h - m) ** 2, axis=0, keepdims=True)
    hn = (h - m) / jnp.sqrt(v + 1e-5) * g1r[...] + be1r[...]
    res = jnp.dot(hn, w2[...], preferred_element_type=jnp.float32)
    y2_o[...] = res[:, :_H]
    r2_o[...] = res[:, _H:]
    cnt_o[...] = c


def _mid(acc, cnt, r1, b1r, g1r, be1r, w2):
    return pl.pallas_call(
        _mid_body,
        out_shape=[
            jax.ShapeDtypeStruct((_N, _H), jnp.float32),
            jax.ShapeDtypeStruct((_N, _H), jnp.float32),
            jax.ShapeDtypeStruct((_N, 1), jnp.float32),
        ],
    )(acc, cnt, r1, b1r, g1r, be1r, w2)


def _bn64(x, g, b):
    m = jnp.mean(x, axis=0, keepdims=True)
    v = jnp.mean((x - m) ** 2, axis=0, keepdims=True)
    return (x - m) / jnp.sqrt(v + 1e-5) * g + b


def _fin_body(acc, cnt, r2, b2r, batch_r, gm1r, bm1r, wl1, bl1, gm2r, bm2r,
              wl2, bl2, gm3r, bm3r, wl3, bl3, wo, bo, out_o):
    c = cnt[...]
    mean = (acc[0, :_N, :] + acc[1, :_N, :]) / jnp.maximum(c, 1.0)
    h2 = jnp.maximum(mean + b2r[...] + r2[...], 0.0)
    gids = lax.broadcasted_iota(jnp.int32, (_G, _N), 0)
    oh = (gids == batch_r[...]).astype(jnp.float32)
    s = jnp.dot(oh, h2, preferred_element_type=jnp.float32)
    cg = jnp.sum(oh, axis=1, keepdims=True)
    xe = s / jnp.maximum(cg, 1.0)

    h = _bn64(xe, gm1r[...], bm1r[...])
    h = jnp.maximum(jnp.dot(h, wl1[...], preferred_element_type=jnp.float32)
                    + bl1[...], 0.0)
    h = _bn64(jnp.concatenate([h, xe], axis=1), gm2r[...], bm2r[...])
    h = jnp.maximum(jnp.dot(h, wl2[...], preferred_element_type=jnp.float32)
                    + bl2[...], 0.0)
    h = _bn64(jnp.concatenate([h, xe], axis=1), gm3r[...], bm3r[...])
    h = jnp.maximum(jnp.dot(h, wl3[...], preferred_element_type=jnp.float32)
                    + bl3[...], 0.0)
    out_o[...] = jnp.dot(h, wo[...], preferred_element_type=jnp.float32) + bo[...]


def _fin(*args):
    return pl.pallas_call(
        _fin_body,
        out_shape=jax.ShapeDtypeStruct((_G, 1), jnp.float32),
    )(*args)


def kernel(x, edge_index, batch, Wl1, Wr1, b1, g1, be1, Wl2, Wr2, b2, gm1, bm1,
           Wlin1, blin1, gm2, bm2, Wlin2, blin2, gm3, bm3, Wlin3, blin3, Wout,
           bout):
    f32 = jnp.float32
    pad = _EPAD - _E
    srcp = jnp.concatenate(
        [edge_index[0], jnp.zeros((pad,), jnp.int32)]).reshape(_NW, _BPW, _EB)
    dstp = jnp.concatenate(
        [edge_index[1], jnp.full((pad,), _N, jnp.int32)]).reshape(_NW, _BPW, _EB)
    z16 = jnp.zeros((_NPAD, _H), f32)
    z1 = jnp.zeros((_NPAD,), f32)

    w1 = jnp.concatenate([Wl1, Wr1], axis=1)
    y1, r1 = _project(x, w1)

    acc1, cnt = _edge_pass(True, y1, srcp, dstp, z16, z1)

    w2 = jnp.concatenate([Wl2, Wr2], axis=1)
    y2, r2, cntc = _mid(acc1, cnt.reshape(_NC, _NPAD, 1), r1,
                        b1.reshape(1, _H), g1.reshape(1, _H),
                        be1.reshape(1, _H), w2)

    acc2 = _edge_pass(False, y2, srcp, dstp, z16, z1)[0]

    out = _fin(acc2, cntc, r2, b2.reshape(1, _H),
               batch.reshape(1, _N).astype(jnp.int32),
               gm1.reshape(1, _H), bm1.reshape(1, _H), Wlin1,
               blin1.reshape(1, _H), gm2.reshape(1, 2 * _H),
               bm2.reshape(1, 2 * _H), Wlin2, blin2.reshape(1, _H),
               gm3.reshape(1, 2 * _H), bm3.reshape(1, 2 * _H), Wlin3,
               blin3.reshape(1, _H), Wout, bout.reshape(1, 1))
    return out


# speedup vs baseline: 47.9649x; 2.0199x over previous
import functools

import jax
import jax.numpy as jnp
from jax import lax
from jax.experimental import pallas as pl
from jax.experimental.pallas import tpu as pltpu
from jax.experimental.pallas import tpu_sc as plsc

_N = 10000
_E = 640000
_D = 128
_H = 16
_G = 64
_EB = 128
_NC = 2
_NS = 16
_NW = _NC * _NS
_BPW = 160
_EPAD = _NW * _BPW * _EB
_NPAD = 10240
_GB = 8
_NGRP = _BPW // _GB
_NPAIR = _NGRP // 2


def _proj_body(x_ref, w_ref, y_ref, r_ref):
    res = jnp.dot(x_ref[...], w_ref[...], preferred_element_type=jnp.float32)
    y_ref[...] = res[:, :_H]
    r_ref[...] = res[:, _H:]


def _project(x, w):
    n, k = x.shape
    bn = 1000
    return pl.pallas_call(
        _proj_body,
        grid=(n // bn,),
        in_specs=[
            pl.BlockSpec((bn, k), lambda i: (i, 0)),
            pl.BlockSpec((k, 2 * _H), lambda i: (0, 0)),
        ],
        out_specs=[
            pl.BlockSpec((bn, _H), lambda i: (i, 0)),
            pl.BlockSpec((bn, _H), lambda i: (i, 0)),
        ],
        out_shape=[
            jax.ShapeDtypeStruct((n, _H), jnp.float32),
            jax.ShapeDtypeStruct((n, _H), jnp.float32),
        ],
    )(x, w)


@functools.lru_cache(maxsize=None)
def _make_edge_pass(with_cnt: bool):
    mesh = plsc.VectorSubcoreMesh(core_axis_name="c", subcore_axis_name="s",
                                  num_cores=_NC, num_subcores=_NS)
    out_type = [jax.ShapeDtypeStruct((_NC, _NPAD, _H), jnp.float32)]
    scratch = [
        pltpu.VMEM_SHARED((_NPAD, _H), jnp.float32),
        pltpu.VMEM((_GB, _EB), jnp.int32),
        pltpu.VMEM((_GB, _EB), jnp.int32),
        pltpu.VMEM((_GB, _EB), jnp.int32),
        pltpu.VMEM((_GB, _EB), jnp.int32),
        pltpu.VMEM((_GB, _EB, _H), jnp.float32),
        pltpu.VMEM((_GB, _EB, _H), jnp.float32),
        pltpu.SemaphoreType.DMA,
        pltpu.SemaphoreType.DMA,
        pltpu.SemaphoreType.DMA,
        pltpu.SemaphoreType.DMA,
    ]
    if with_cnt:
        out_type.append(jax.ShapeDtypeStruct((_NC, _NPAD), jnp.float32))
        scratch += [
            pltpu.VMEM_SHARED((_NPAD,), jnp.float32),
            pltpu.VMEM((_EB,), jnp.float32),
        ]
    st = _NPAD // _NS

    def body(y_hbm, src_hbm, dst_hbm, z_hbm, z1_hbm, *rest):
        if with_cnt:
            (acc_out, cnt_out, acc_sh, sbufA, sbufB, dbufA, dbufB, rowsA,
             rowsB, semGA, semGB, semS, semSL, cnt_sh, ones_v) = rest
        else:
            (acc_out, acc_sh, sbufA, sbufB, dbufA, dbufB, rowsA, rowsB,
             semGA, semGB, semS, semSL) = rest
        cid = lax.axis_index("c")
        sid = lax.axis_index("s")
        wid = sid * _NC + cid

        pltpu.sync_copy(z_hbm.at[pl.ds(sid * st, st)],
                        acc_sh.at[pl.ds(sid * st, st)])
        if with_cnt:
            pltpu.sync_copy(z1_hbm.at[pl.ds(sid * st, st)],
                            cnt_sh.at[pl.ds(sid * st, st)])
            for i in range(_EB // 16):
                ones_v[pl.ds(i * 16, 16)] = jnp.ones((16,), jnp.float32)

        def load_slabs_sync(g, sb, db):
            pltpu.sync_copy(src_hbm.at[wid, pl.ds(g * _GB, _GB)], sb)
            pltpu.sync_copy(dst_hbm.at[wid, pl.ds(g * _GB, _GB)], db)

        def fire_gathers(sb, rows, sem):
            for b in range(_GB):
                pltpu.async_copy(y_hbm.at[sb.at[b]], rows.at[b], sem)

        def drain_gathers(sb, rows, sem):
            for b in range(_GB):
                pltpu.make_async_copy(y_hbm.at[sb.at[b]], rows.at[b],
                                      sem).wait()

        def fire_scatters(db, rows):
            for b in range(_GB):
                pltpu.async_copy(rows.at[b], acc_sh.at[db.at[b]], semS,
                                 add=True)

        def drain_scatters(db, rows):
            for b in range(_GB):
                pltpu.make_async_copy(rows.at[b], acc_sh.at[db.at[b]],
                                      semS).wait()

        plsc.subcore_barrier()

        def fire_scatters(db, rows):
            for b in range(_GB):
                pltpu.async_copy(rows.at[b], acc_sh.at[db.at[b]], semS,
                                 add=True)
                if with_cnt:
                    pltpu.async_copy(ones_v, cnt_sh.at[db.at[b]], semS,
                                     add=True)

        def drain_scatters(db, rows):
            for b in range(_GB):
                pltpu.make_async_copy(rows.at[b], acc_sh.at[db.at[b]],
                                      semS).wait()
                if with_cnt:
                    pltpu.make_async_copy(ones_v, cnt_sh.at[db.at[b]],
                                          semS).wait()

        load_slabs_sync(0, sbufA, dbufA)
        fire_gathers(sbufA, rowsA, semGA)
        load_slabs_sync(1, sbufB, dbufB)

        def pair(t, carry):
            drain_gathers(sbufA, rowsA, semGA)
            fire_gathers(sbufB, rowsB, semGB)
            fire_scatters(dbufA, rowsA)
            drain_scatters(dbufA, rowsA)

            @pl.when(t < _NPAIR - 1)
            def _():
                pltpu.async_copy(src_hbm.at[wid, pl.ds((2 * t + 2) * _GB, _GB)],
                                 sbufA, semSL)
                pltpu.async_copy(dst_hbm.at[wid, pl.ds((2 * t + 2) * _GB, _GB)],
                                 dbufA, semSL)

            drain_gathers(sbufB, rowsB, semGB)
            fire_scatters(dbufB, rowsB)
            drain_scatters(dbufB, rowsB)

            @pl.when(t < _NPAIR - 1)
            def _():
                pltpu.make_async_copy(src_hbm.at[wid, pl.ds(0, _GB)], sbufA,
                                      semSL).wait()
                pltpu.make_async_copy(dst_hbm.at[wid, pl.ds(0, _GB)], dbufA,
                                      semSL).wait()
                fire_gathers(sbufA, rowsA, semGA)
                load_slabs_sync(2 * t + 3, sbufB, dbufB)
            return carry

        lax.fori_loop(0, _NPAIR, pair, 0)
        plsc.subcore_barrier()

        pltpu.sync_copy(acc_sh.at[pl.ds(sid * st, st)],
                        acc_out.at[cid, pl.ds(sid * st, st)])
        if with_cnt:
            pltpu.sync_copy(cnt_sh.at[pl.ds(sid * st, st)],
                            cnt_out.at[cid, pl.ds(sid * st, st)])

    return pl.kernel(
        body, out_type=out_type, mesh=mesh, scratch_types=scratch,
        compiler_params=pltpu.CompilerParams(use_tc_tiling_on_sc=False))


def _edge_pass(with_cnt, *args):
    return _make_edge_pass(with_cnt)(*args)


def _mid_body(acc, cnt, r1, b1r, g1r, be1r, w2, y2_o, r2_o, cnt_o):
    s = acc[0, :_N, :] + acc[1, :_N, :]
    c = cnt[0, :_N, :] + cnt[1, :_N, :]
    mean = s / jnp.maximum(c, 1.0)
    h = jnp.maximum(mean + b1r[...] + r1[...], 0.0)
    m = jnp.mean(h, axis=0, keepdims=True)
    v = jnp.mean((h - m) ** 2, axis=0, keepdims=True)
    hn = (h - m) / jnp.sqrt(v + 1e-5) * g1r[...] + be1r[...]
    res = jnp.dot(hn, w2[...], preferred_element_type=jnp.float32)
    y2_o[...] = res[:, :_H]
    r2_o[...] = res[:, _H:]
    cnt_o[...] = c


def _mid(acc, cnt, r1, b1r, g1r, be1r, w2):
    return pl.pallas_call(
        _mid_body,
        out_shape=[
            jax.ShapeDtypeStruct((_N, _H), jnp.float32),
            jax.ShapeDtypeStruct((_N, _H), jnp.float32),
            jax.ShapeDtypeStruct((_N, 1), jnp.float32),
        ],
    )(acc, cnt, r1, b1r, g1r, be1r, w2)


def _bn64(x, g, b):
    m = jnp.mean(x, axis=0, keepdims=True)
    v = jnp.mean((x - m) ** 2, axis=0, keepdims=True)
    return (x - m) / jnp.sqrt(v + 1e-5) * g + b


def _fin_body(acc, cnt, r2, b2r, batch_r, gm1r, bm1r, wl1, bl1, gm2r, bm2r,
              wl2, bl2, gm3r, bm3r, wl3, bl3, wo, bo, out_o):
    c = cnt[...]
    mean = (acc[0, :_N, :] + acc[1, :_N, :]) / jnp.maximum(c, 1.0)
    h2 = jnp.maximum(mean + b2r[...] + r2[...], 0.0)
    gids = lax.broadcasted_iota(jnp.int32, (_G, _N), 0)
    oh = (gids == batch_r[...]).astype(jnp.float32)
    s = jnp.dot(oh, h2, preferred_element_type=jnp.float32)
    cg = jnp.sum(oh, axis=1, keepdims=True)
    xe = s / jnp.maximum(cg, 1.0)

    h = _bn64(xe, gm1r[...], bm1r[...])
    h = jnp.maximum(jnp.dot(h, wl1[...], preferred_element_type=jnp.float32)
                    + bl1[...], 0.0)
    h = _bn64(jnp.concatenate([h, xe], axis=1), gm2r[...], bm2r[...])
    h = jnp.maximum(jnp.dot(h, wl2[...], preferred_element_type=jnp.float32)
                    + bl2[...], 0.0)
    h = _bn64(jnp.concatenate([h, xe], axis=1), gm3r[...], bm3r[...])
    h = jnp.maximum(jnp.dot(h, wl3[...], preferred_element_type=jnp.float32)
                    + bl3[...], 0.0)
    out_o[...] = jnp.dot(h, wo[...], preferred_element_type=jnp.float32) + bo[...]


def _fin(*args):
    return pl.pallas_call(
        _fin_body,
        out_shape=jax.ShapeDtypeStruct((_G, 1), jnp.float32),
    )(*args)


def kernel(x, edge_index, batch, Wl1, Wr1, b1, g1, be1, Wl2, Wr2, b2, gm1, bm1,
           Wlin1, blin1, gm2, bm2, Wlin2, blin2, gm3, bm3, Wlin3, blin3, Wout,
           bout):
    f32 = jnp.float32
    pad = _EPAD - _E
    pad_iota = lax.iota(jnp.int32, pad)
    srcp = jnp.concatenate(
        [edge_index[0], pad_iota % _N]).reshape(_NW, _BPW, _EB)
    dstp = jnp.concatenate(
        [edge_index[1], _N + pad_iota % (_NPAD - _N)]).reshape(_NW, _BPW, _EB)
    z16 = jnp.zeros((_NPAD, _H), f32)
    z1 = jnp.zeros((_NPAD,), f32)

    w1 = jnp.concatenate([Wl1, Wr1], axis=1)
    y1, r1 = _project(x, w1)

    acc1, cnt = _edge_pass(True, y1, srcp, dstp, z16, z1)

    w2 = jnp.concatenate([Wl2, Wr2], axis=1)
    y2, r2, cntc = _mid(acc1, cnt.reshape(_NC, _NPAD, 1), r1,
                        b1.reshape(1, _H), g1.reshape(1, _H),
                        be1.reshape(1, _H), w2)

    acc2 = _edge_pass(False, y2, srcp, dstp, z16, z1)[0]

    out = _fin(acc2, cntc, r2, b2.reshape(1, _H),
               batch.reshape(1, _N).astype(jnp.int32),
               gm1.reshape(1, _H), bm1.reshape(1, _H), Wlin1,
               blin1.reshape(1, _H), gm2.reshape(1, 2 * _H),
               bm2.reshape(1, 2 * _H), Wlin2, blin2.reshape(1, _H),
               gm3.reshape(1, 2 * _H), bm3.reshape(1, 2 * _H), Wlin3,
               blin3.reshape(1, _H), Wout, bout.reshape(1, 1))
    return out
